# Initial kernel scaffold; baseline (speedup 1.0000x reference)
import jax
import jax.numpy as jnp
from jax.experimental import pallas as pl


def kernel(x, slab):
    raise NotImplementedError("write your pallas kernel here")


# speedup vs baseline: 1.0230x; 1.0230x over previous
import jax
import jax.numpy as jnp
from jax.experimental import pallas as pl
from jax.experimental.pallas import tpu as pltpu

_HID = 32
_PACK = 4
_IN = 4
_ACT = 2
_KIN = _PACK * _IN
_NOUT = _PACK * _ACT

_OFF_BIAS = 0
_OFF_W1 = 8
_OFF_W2 = _OFF_W1 + _KIN
_OFF_W3 = _OFF_W2 + 128
_PROWS = _OFF_W3 + 128


def _prep(slab):
    w1 = slab[0:_IN, 0:_HID]
    w2 = slab[16:16 + _HID, 0:_HID]
    w3 = slab[48:48 + _HID, 0:_ACT]
    b1 = slab[8, 0:_HID]
    b2 = slab[9, 0:_HID]
    b3 = slab[10, 0:_ACT]

    blk = jax.scipy.linalg.block_diag
    p = jnp.zeros((_PROWS, 128), jnp.float32)
    p = p.at[_OFF_BIAS + 0, :].set(jnp.tile(b1, _PACK))
    p = p.at[_OFF_BIAS + 1, :].set(jnp.tile(b2, _PACK))
    p = p.at[_OFF_BIAS + 2, :_NOUT].set(jnp.tile(b3, _PACK))
    p = p.at[_OFF_W1:_OFF_W1 + _KIN, :].set(blk(*([w1] * _PACK)))
    p = p.at[_OFF_W2:_OFF_W2 + 128, :].set(blk(*([w2] * _PACK)))
    p = p.at[_OFF_W3:_OFF_W3 + 128, :_NOUT].set(blk(*([w3] * _PACK)))
    return p


def _mlp_body(x_ref, p_ref, q_ref):
    xv = x_ref[...]
    b1 = p_ref[_OFF_BIAS + 0:_OFF_BIAS + 1, :]
    b2 = p_ref[_OFF_BIAS + 1:_OFF_BIAS + 2, :]
    b3 = p_ref[_OFF_BIAS + 2:_OFF_BIAS + 3, :_NOUT]
    w1 = p_ref[_OFF_W1:_OFF_W1 + _KIN, :]
    w2 = p_ref[_OFF_W2:_OFF_W2 + 128, :]
    w3 = p_ref[_OFF_W3:_OFF_W3 + 128, :_NOUT]

    h1 = jnp.maximum(
        jnp.dot(xv, w1, preferred_element_type=jnp.float32) + b1, 0.0)
    h2 = jnp.maximum(
        jnp.dot(h1, w2, preferred_element_type=jnp.float32) + b2, 0.0)
    q_ref[...] = jnp.dot(h2, w3, preferred_element_type=jnp.float32) + b3


def kernel(x, slab):
    batch = x.shape[0]
    rows = batch // _PACK
    xp = x.reshape(rows, _KIN)
    p = _prep(slab)

    tile = 2048
    while rows % tile:
        tile //= 2
    grid = rows // tile

    flops = 2 * batch * (_IN * _HID + _HID * _HID + _HID * _ACT)
    cost = pl.CostEstimate(
        flops=flops, transcendentals=0,
        bytes_accessed=x.size * 4 + _PROWS * 128 * 4 + batch * _ACT * 4)

    qp = pl.pallas_call(
        _mlp_body,
        out_shape=jax.ShapeDtypeStruct((rows, _NOUT), jnp.float32),
        grid=(grid,),
        in_specs=[pl.BlockSpec((tile, _KIN), lambda i: (i, 0)),
                  pl.BlockSpec((_PROWS, 128), lambda i: (0, 0))],
        out_specs=pl.BlockSpec((tile, _NOUT), lambda i: (i, 0)),
        compiler_params=pltpu.CompilerParams(
            dimension_semantics=("parallel",)),
        cost_estimate=cost,
    )(xp, p)
    return qp.reshape(batch, _ACT)
